# baseline (speedup 1.0000x reference)
import functools

import jax
import jax.numpy as jnp
from jax import lax
from jax.experimental import pallas as pl
from jax.experimental.pallas import tpu as pltpu
from jax.experimental.pallas import tpu_sc as plsc

N_NODE = 50000
EMB = 64
BATCH = 512
SEQ = 50
N_EDGES = 800000
SOFT_LAMBDA = 0.2
W_K = 10.0

_D = EMB // 2
_ACC_ROWS = 50176
_ZQ = 3136
_CH = 128
_NCHUNK = 6400
_E_PAD = _NCHUNK * _CH
_NCH_TILE = _NCHUNK // 16
_SUP = 8
_NSUP = _NCH_TILE // _SUP
_IPN = _SUP // 4
_JPN = _NSUP // 2


def _segsum_body(h_hbm, packed_hbm, vals_hbm, out_hbm, acc,
                 pb0, pb1, vb0, vb1, rb0, rb1, rb2, rb3, li0, li1, li2, li3,
                 cb0, cb1, cb2, cb3,
                 gs0, gs1, gs2, gs3, ss0, ss1, ss2, ss3, ps0, ps1):
    c = lax.axis_index("c")
    s = lax.axis_index("s")
    coff = c * N_NODE
    pb = (pb0, pb1)
    vb = (vb0, vb1)
    rb = (rb0, rb1, rb2, rb3)
    li = (li0, li1, li2, li3)
    cb = (cb0, cb1, cb2, cb3)
    gs = (gs0, gs1, gs2, gs3)
    ss = (ss0, ss1, ss2, ss3)
    ps = (ps0, ps1)
    cstart = s * _NCH_TILE

    @plsc.parallel_loop(0, _CH)
    def _(i):
        for d in range(_D // 16):
            rb0[i, pl.ds(16 * d, 16)] = jnp.zeros((16,), jnp.float32)
    arow = s * _ZQ
    for j in range(24):
        pltpu.sync_copy(rb0, acc.at[pl.ds(arow + j * _CH, _CH)])
    pltpu.sync_copy(rb0.at[pl.ds(0, 64)], acc.at[pl.ds(arow + 24 * _CH, 64)])
    plsc.subcore_barrier()

    def prep_gather(pbuf, ic, b):
        cbb = cb[b]
        for g in range(_CH // 16):
            cbb[pl.ds(16 * g, 16)] = pbuf[ic, 0, pl.ds(16 * g, 16)] + coff

    def issue_gather(b):
        pltpu.make_async_copy(h_hbm.at[cb[b]], rb[b], gs[b]).start()

    def wait_gather(b):
        pltpu.make_async_copy(h_hbm.at[cb[b]], rb[b], gs[b]).wait()

    def issue_scatter(b):
        pltpu.make_async_copy(rb[b], acc.at[li[b]], ss[b]).start(add=True)

    def wait_scatter(b):
        pltpu.make_async_copy(rb[b], acc.at[li[b]], ss[b]).wait()

    def process(pbuf, vbuf, ic, b):
        lib = li[b]
        rbb = rb[b]
        for g in range(_CH // 16):
            lib[pl.ds(16 * g, 16)] = pbuf[ic, 1, pl.ds(16 * g, 16)]

        @plsc.parallel_loop(0, _CH // 16, unroll=2)
        def _(g):
            val16 = vbuf[ic, 0, pl.ds(g * 16, 16)]
            for lane in range(16):
                e = g * 16 + lane
                v = val16[lane]
                for d in range(_D // 16):
                    rbb[e, pl.ds(16 * d, 16)] = rbb[e, pl.ds(16 * d, 16)] * v

    pltpu.sync_copy(packed_hbm.at[pl.ds(cstart, _SUP)], pb0)
    pltpu.sync_copy(vals_hbm.at[pl.ds(cstart, _SUP)], vb0)
    pltpu.make_async_copy(packed_hbm.at[pl.ds(cstart + _SUP, _SUP)], pb1, ps1).start()
    pltpu.make_async_copy(vals_hbm.at[pl.ds(cstart + _SUP, _SUP)], vb1, ps1).start()
    prep_gather(pb0, 0, 0)
    issue_gather(0)
    prep_gather(pb0, 1, 1)
    issue_gather(1)

    def make_super(J2):
        nb = 1 - J2
        pbuf = pb[J2]
        npbuf = pb[nb]
        vbuf = vb[J2]
        nvbuf = vb[nb]

        def super_body(jp):
            j = 2 * jp + J2
            sbase = cstart + j * _SUP
            pref_cond = (jp > 0) if J2 == 0 else (jp < _JPN - 1)

            @pl.when(pref_cond)
            def _():
                pltpu.make_async_copy(
                    packed_hbm.at[pl.ds(sbase + _SUP, _SUP)], npbuf, ps[nb]).start()
                pltpu.make_async_copy(
                    vals_hbm.at[pl.ds(sbase + _SUP, _SUP)], nvbuf, ps[nb]).start()

            def pair_body(ip, carry):
                for b in range(4):
                    ic = 4 * ip + b
                    wait_gather(b)
                    process(pbuf, vbuf, ic, b)
                    issue_scatter(b)
                    b2 = (b + 2) % 4
                    if b < 2:
                        prep_gather(pbuf, ic + 2, b2)
                        if J2 == 0:
                            @pl.when((jp > 0) | (ip > 0))
                            def _():
                                wait_scatter(b2)
                        else:
                            wait_scatter(b2)
                        issue_gather(b2)
                    else:
                        @pl.when(ip < _IPN - 1)
                        def _():
                            prep_gather(pbuf, ic + 2, b2)
                            wait_scatter(b2)
                            issue_gather(b2)

                        cond = ((ip == _IPN - 1) if J2 == 0
                                else ((ip == _IPN - 1) & (jp < _JPN - 1)))

                        @pl.when(cond)
                        def _():
                            if b == 2:
                                pltpu.make_async_copy(
                                    packed_hbm.at[pl.ds(cstart, _SUP)],
                                    npbuf, ps[nb]).wait()
                                pltpu.make_async_copy(
                                    vals_hbm.at[pl.ds(cstart, _SUP)],
                                    nvbuf, ps[nb]).wait()
                            prep_gather(npbuf, b - 2, b2)
                            wait_scatter(b2)
                            issue_gather(b2)
                return carry

            lax.fori_loop(0, _IPN, pair_body, 0)

        return super_body

    sup0 = make_super(0)
    sup1 = make_super(1)

    def outer(jp, carry):
        sup0(jp)
        sup1(jp)
        return carry

    lax.fori_loop(0, _JPN, outer, 0)

    for b in range(4):
        wait_scatter(b)
    plsc.subcore_barrier()

    dstart = s * _ZQ

    @pl.when(s < 15)
    def _():
        pltpu.sync_copy(acc.at[pl.ds(dstart, _ZQ)],
                        out_hbm.at[c, pl.ds(dstart, _ZQ)])

    @pl.when(s == 15)
    def _():
        pltpu.sync_copy(acc.at[pl.ds(dstart, 2960)],
                        out_hbm.at[c, pl.ds(dstart, 2960)])


@functools.cache
def _make_segsum_call():
  return pl.kernel(
    _segsum_body,
    out_type=jax.ShapeDtypeStruct((2, N_NODE, _D), jnp.float32),
    mesh=plsc.VectorSubcoreMesh(core_axis_name="c", subcore_axis_name="s",
                                num_cores=2, num_subcores=16),
    compiler_params=pltpu.CompilerParams(use_tc_tiling_on_sc=False),
    scratch_types=[
        pltpu.VMEM_SHARED((_ACC_ROWS, _D), jnp.float32),
        pltpu.VMEM((_SUP, 2, _CH), jnp.int32),
        pltpu.VMEM((_SUP, 2, _CH), jnp.int32),
        pltpu.VMEM((_SUP, 1, _CH), jnp.float32),
        pltpu.VMEM((_SUP, 1, _CH), jnp.float32),
        pltpu.VMEM((_CH, _D), jnp.float32),
        pltpu.VMEM((_CH, _D), jnp.float32),
        pltpu.VMEM((_CH, _D), jnp.float32),
        pltpu.VMEM((_CH, _D), jnp.float32),
        pltpu.VMEM((_CH,), jnp.int32),
        pltpu.VMEM((_CH,), jnp.int32),
        pltpu.VMEM((_CH,), jnp.int32),
        pltpu.VMEM((_CH,), jnp.int32),
        pltpu.VMEM((_CH,), jnp.int32),
        pltpu.VMEM((_CH,), jnp.int32),
        pltpu.VMEM((_CH,), jnp.int32),
        pltpu.VMEM((_CH,), jnp.int32),
        pltpu.SemaphoreType.DMA,
        pltpu.SemaphoreType.DMA,
        pltpu.SemaphoreType.DMA,
        pltpu.SemaphoreType.DMA,
        pltpu.SemaphoreType.DMA,
        pltpu.SemaphoreType.DMA,
        pltpu.SemaphoreType.DMA,
        pltpu.SemaphoreType.DMA,
        pltpu.SemaphoreType.DMA,
        pltpu.SemaphoreType.DMA,
    ],
  )


def _pack_edges(adj_row, adj_col, adj_val):
    pad = _E_PAD - N_EDGES
    r = jnp.concatenate([adj_row, jnp.zeros((pad,), adj_row.dtype)])
    col = jnp.concatenate([adj_col, jnp.zeros((pad,), adj_col.dtype)])
    v = jnp.concatenate([adj_val, jnp.zeros((pad,), adj_val.dtype)])
    packed = jnp.stack([col.reshape(_NCHUNK, _CH).astype(jnp.int32),
                        r.reshape(_NCHUNK, _CH).astype(jnp.int32)], axis=1)
    return packed, v.reshape(_NCHUNK, 1, _CH)


def _segsum(h, packed, vals):
    hcat = jnp.concatenate([h[:, :_D], h[:, _D:]], axis=0)
    seg = _make_segsum_call()(hcat, packed, vals)
    return jnp.concatenate([seg[0], seg[1]], axis=-1)


def _l2norm(x, axis):
    n = jnp.sqrt(jnp.sum(x * x, axis=axis, keepdims=True))
    return x / jnp.maximum(n, 1e-12)


_BLK = 2000


def _mm(a, b):
    return lax.dot_general(a, b, (((1,), (0,)), ((), ())),
                           preferred_element_type=jnp.float32)


def _pre_body(e_ref, img_ref, txt_ref, wpi_ref, bpi_ref, wpt_ref, bpt_ref,
              w0_ref, pi_ref, pt_ref, h1_ref, gi_ref, gt_ref):
    i = pl.program_id(0)
    e = e_ref[...]
    pi = _mm(e, wpi_ref[...]) + bpi_ref[...][None, :]
    pt = _mm(e, wpt_ref[...]) + bpt_ref[...][None, :]
    pi_ref[...] = pi
    pt_ref[...] = pt
    h1_ref[...] = _mm(e, w0_ref[...])
    gi_p = lax.dot_general(pi, img_ref[...], (((0,), (0,)), ((), ())),
                           preferred_element_type=jnp.float32)
    gt_p = lax.dot_general(pt, txt_ref[...], (((0,), (0,)), ((), ())),
                           preferred_element_type=jnp.float32)

    @pl.when(i == 0)
    def _():
        gi_ref[...] = jnp.zeros_like(gi_ref)
        gt_ref[...] = jnp.zeros_like(gt_ref)

    gi_ref[...] += gi_p
    gt_ref[...] += gt_p


def _pre_stage(embedding, image_table, text_table, prompt_img_W, prompt_img_b,
               prompt_txt_W, prompt_txt_b, w_item0):
    nblk = pl.BlockSpec((_BLK, EMB), lambda i: (i, 0))
    wspec = pl.BlockSpec((EMB, EMB), lambda i: (0, 0))
    bspec = pl.BlockSpec((EMB,), lambda i: (0,))
    gspec = pl.BlockSpec((EMB, EMB), lambda i: (0, 0))
    return pl.pallas_call(
        _pre_body,
        grid=(N_NODE // _BLK,),
        in_specs=[nblk, nblk, nblk, wspec, bspec, wspec, bspec, wspec],
        out_specs=[nblk, nblk, nblk, gspec, gspec],
        out_shape=[
            jax.ShapeDtypeStruct((N_NODE, EMB), jnp.float32),
            jax.ShapeDtypeStruct((N_NODE, EMB), jnp.float32),
            jax.ShapeDtypeStruct((N_NODE, EMB), jnp.float32),
            jax.ShapeDtypeStruct((EMB, EMB), jnp.float32),
            jax.ShapeDtypeStruct((EMB, EMB), jnp.float32),
        ],
    )(embedding, image_table, text_table, prompt_img_W, prompt_img_b,
      prompt_txt_W, prompt_txt_b, w_item0)


def _mid_body(seg1_ref, w1_ref, h2_ref, n1_ref):
    seg1 = seg1_ref[...]
    h2_ref[...] = _mm(seg1, w1_ref[...])
    n1_ref[...] = _l2norm(seg1, -1)


def _mid_stage(seg1, w_item1):
    nblk = pl.BlockSpec((_BLK, EMB), lambda i: (i, 0))
    wspec = pl.BlockSpec((EMB, EMB), lambda i: (0, 0))
    return pl.pallas_call(
        _mid_body,
        grid=(N_NODE // _BLK,),
        in_specs=[nblk, wspec],
        out_specs=[nblk, nblk],
        out_shape=[
            jax.ShapeDtypeStruct((N_NODE, EMB), jnp.float32),
            jax.ShapeDtypeStruct((N_NODE, EMB), jnp.float32),
        ],
    )(seg1, w_item1)


def _re_body(img_ref, txt_ref, pi_ref, pt_ref, gi_ref, gt_ref, w1_ref,
             b1_ref, racc_ref):
    re_img = img_ref[...] + SOFT_LAMBDA * _l2norm(_mm(pi_ref[...], gi_ref[...]), -1)
    re_txt = txt_ref[...] + SOFT_LAMBDA * _l2norm(_mm(pt_ref[...], gt_ref[...]), -1)
    w1 = w1_ref[...]
    racc_ref[...] = (_mm(re_img, w1[EMB:2 * EMB])
                     + _mm(re_txt, w1[2 * EMB:3 * EMB]) + b1_ref[...][None, :])


def _re_stage(image_table, text_table, pi, pt, gi, gt, mlp1_W, mlp1_b):
    nblk = pl.BlockSpec((_BLK, EMB), lambda i: (i, 0))
    gspec = pl.BlockSpec((EMB, EMB), lambda i: (0, 0))
    w1spec = pl.BlockSpec((3 * EMB, EMB), lambda i: (0, 0))
    bspec = pl.BlockSpec((EMB,), lambda i: (0,))
    return pl.pallas_call(
        _re_body,
        grid=(N_NODE // _BLK,),
        in_specs=[nblk, nblk, nblk, nblk, gspec, gspec, w1spec, bspec],
        out_specs=nblk,
        out_shape=jax.ShapeDtypeStruct((N_NODE, EMB), jnp.float32),
    )(image_table, text_table, pi, pt, gi, gt, mlp1_W, mlp1_b)


def _mix_body(e_ref, n1_ref, seg2_ref, racc_ref, w1_ref, w2_ref, b2_ref,
              mixed_ref):
    item = (e_ref[...] + n1_ref[...] + _l2norm(seg2_ref[...], -1)) * (1.0 / 3.0)
    z = jnp.tanh(_mm(item, w1_ref[...][0:EMB]) + racc_ref[...])
    mixed_ref[...] = jnp.tanh(_mm(z, w2_ref[...]) + b2_ref[...][None, :])


def _mix_stage(embedding, n1, seg2, racc, mlp1_W, mlp2_W, mlp2_b):
    nblk = pl.BlockSpec((_BLK, EMB), lambda i: (i, 0))
    gspec = pl.BlockSpec((EMB, EMB), lambda i: (0, 0))
    w1spec = pl.BlockSpec((3 * EMB, EMB), lambda i: (0, 0))
    bspec = pl.BlockSpec((EMB,), lambda i: (0,))
    return pl.pallas_call(
        _mix_body,
        grid=(N_NODE // _BLK,),
        in_specs=[nblk, nblk, nblk, nblk, w1spec, gspec, bspec],
        out_specs=nblk,
        out_shape=jax.ShapeDtypeStruct((N_NODE, EMB), jnp.float32),
    )(embedding, n1, seg2, racc, mlp1_W, mlp2_W, mlp2_b)


def _scores_body(select_ref, mixed_ref, out_ref):
    out_ref[...] = W_K * jax.lax.dot_general(
        select_ref[...], mixed_ref[...],
        (((1,), (1,)), ((), ())),
        preferred_element_type=jnp.float32)


def _scores_matmul(select, mixed):
    blk = 2048
    grid = (pl.cdiv(N_NODE, blk),)
    return pl.pallas_call(
        _scores_body,
        grid=grid,
        in_specs=[
            pl.BlockSpec((BATCH, EMB), lambda i: (0, 0)),
            pl.BlockSpec((blk, EMB), lambda i: (i, 0)),
        ],
        out_specs=pl.BlockSpec((BATCH, blk), lambda i: (0, i)),
        out_shape=jax.ShapeDtypeStruct((BATCH, N_NODE), jnp.float32),
    )(select, mixed)


def kernel(session_item, session_len, reversed_sess_item, mask, tar, adj_row, adj_col, adj_val, embedding, pos_embedding, image_table, text_table, w_item0, w_item1, w_1, w_2, glu1_W, glu1_b, glu2_W, prompt_img_W, prompt_img_b, prompt_txt_W, prompt_txt_b, mlp1_W, mlp1_b, mlp2_W, mlp2_b):
    pi, pt, h1pre, gi, gt = _pre_stage(
        embedding, image_table, text_table, prompt_img_W, prompt_img_b,
        prompt_txt_W, prompt_txt_b, w_item0)
    packed, vals = _pack_edges(adj_row, adj_col, adj_val)
    seg1 = _segsum(h1pre, packed, vals)
    racc = _re_stage(image_table, text_table, pi, pt, gi, gt, mlp1_W, mlp1_b)
    h2pre, n1 = _mid_stage(seg1, w_item1)
    seg2 = _segsum(h2pre, packed, vals)
    mixed = _mix_stage(embedding, n1, seg2, racc, mlp1_W, mlp2_W, mlp2_b)
    table = jnp.concatenate([jnp.zeros((1, EMB), jnp.float32), mixed], axis=0)
    seq_h = jnp.take(table, reversed_sess_item, axis=0)
    hs = jnp.sum(seq_h, axis=1) / session_len
    m = mask.astype(jnp.float32)[..., None]
    pos = jnp.broadcast_to(pos_embedding[:SEQ][None, :, :], seq_h.shape)
    nh = jnp.tanh(jnp.concatenate([pos, seq_h], axis=-1) @ w_1)
    nh = jax.nn.sigmoid(nh @ glu1_W + glu1_b + hs[:, None, :] @ glu2_W)
    beta = (nh @ w_2) * m
    select = jnp.sum(beta * seq_h, axis=1)
    return _scores_matmul(select, mixed)


# speedup vs baseline: 1.1528x; 1.1528x over previous
import functools

import jax
import jax.numpy as jnp
from jax import lax
from jax.experimental import pallas as pl
from jax.experimental.pallas import tpu as pltpu
from jax.experimental.pallas import tpu_sc as plsc

N_NODE = 50000
EMB = 64
BATCH = 512
SEQ = 50
N_EDGES = 800000
SOFT_LAMBDA = 0.2
W_K = 10.0

_D = EMB // 2
_ACC_ROWS = 50176
_ZQ = 3136
_CH = 128
_NCHUNK = 6400
_E_PAD = _NCHUNK * _CH
_NCH_TILE = _NCHUNK // 16
_SUP = 8
_NSUP = _NCH_TILE // _SUP
_IPN = _SUP // 4
_JPN = _NSUP // 2


def _segsum_body(h0_hbm, h1_hbm, packed_hbm, vals_hbm, out_hbm, acc,
                 pb0, pb1, vb0, vb1, rb0, rb1, rb2, rb3, li0, li1, li2, li3,
                 gs0, gs1, gs2, gs3, ss0, ss1, ss2, ss3, ps0, ps1):
    c = lax.axis_index("c")
    s = lax.axis_index("s")
    pb = (pb0, pb1)
    vb = (vb0, vb1)
    rb = (rb0, rb1, rb2, rb3)
    li = (li0, li1, li2, li3)
    gs = (gs0, gs1, gs2, gs3)
    ss = (ss0, ss1, ss2, ss3)
    ps = (ps0, ps1)
    cstart = s * _NCH_TILE

    @plsc.parallel_loop(0, _CH)
    def _(i):
        for d in range(_D // 16):
            rb0[i, pl.ds(16 * d, 16)] = jnp.zeros((16,), jnp.float32)
    arow = s * _ZQ
    for j in range(24):
        pltpu.sync_copy(rb0, acc.at[pl.ds(arow + j * _CH, _CH)])
    pltpu.sync_copy(rb0.at[pl.ds(0, 64)], acc.at[pl.ds(arow + 24 * _CH, 64)])
    plsc.subcore_barrier()

    def issue_gather(pbuf, ic, b):
        @pl.when(c == 0)
        def _():
            pltpu.make_async_copy(h0_hbm.at[pbuf.at[ic, 0]], rb[b], gs[b]).start()

        @pl.when(c == 1)
        def _():
            pltpu.make_async_copy(h1_hbm.at[pbuf.at[ic, 0]], rb[b], gs[b]).start()

    def wait_gather(b):
        pltpu.make_async_copy(h0_hbm.at[pb0.at[0, 0]], rb[b], gs[b]).wait()

    def issue_scatter(b):
        pltpu.make_async_copy(rb[b], acc.at[li[b]], ss[b]).start(add=True)

    def wait_scatter(b):
        pltpu.make_async_copy(rb[b], acc.at[li[b]], ss[b]).wait()

    def process(pbuf, vbuf, ic, b):
        lib = li[b]
        rbb = rb[b]
        for g in range(_CH // 16):
            lib[pl.ds(16 * g, 16)] = pbuf[ic, 1, pl.ds(16 * g, 16)]

        @plsc.parallel_loop(0, _CH // 16, unroll=2)
        def _(g):
            val16 = vbuf[ic, 0, pl.ds(g * 16, 16)]
            for lane in range(16):
                e = g * 16 + lane
                v = val16[lane]
                for d in range(_D // 16):
                    rbb[e, pl.ds(16 * d, 16)] = rbb[e, pl.ds(16 * d, 16)] * v

    pltpu.sync_copy(packed_hbm.at[pl.ds(cstart, _SUP)], pb0)
    pltpu.sync_copy(vals_hbm.at[pl.ds(cstart, _SUP)], vb0)
    pltpu.make_async_copy(packed_hbm.at[pl.ds(cstart + _SUP, _SUP)], pb1, ps1).start()
    pltpu.make_async_copy(vals_hbm.at[pl.ds(cstart + _SUP, _SUP)], vb1, ps1).start()
    issue_gather(pb0, 0, 0)
    issue_gather(pb0, 1, 1)

    def make_super(J2):
        nb = 1 - J2
        pbuf = pb[J2]
        npbuf = pb[nb]
        vbuf = vb[J2]
        nvbuf = vb[nb]

        def super_body(jp):
            j = 2 * jp + J2
            sbase = cstart + j * _SUP
            pref_cond = (jp > 0) if J2 == 0 else (jp < _JPN - 1)

            @pl.when(pref_cond)
            def _():
                pltpu.make_async_copy(
                    packed_hbm.at[pl.ds(sbase + _SUP, _SUP)], npbuf, ps[nb]).start()
                pltpu.make_async_copy(
                    vals_hbm.at[pl.ds(sbase + _SUP, _SUP)], nvbuf, ps[nb]).start()

            def pair_body(ip, carry):
                for b in range(4):
                    ic = 4 * ip + b
                    wait_gather(b)
                    process(pbuf, vbuf, ic, b)
                    issue_scatter(b)
                    b2 = (b + 2) % 4
                    if b < 2:
                        if J2 == 0:
                            @pl.when((jp > 0) | (ip > 0))
                            def _():
                                wait_scatter(b2)
                        else:
                            wait_scatter(b2)
                        issue_gather(pbuf, ic + 2, b2)
                    else:
                        @pl.when(ip < _IPN - 1)
                        def _():
                            wait_scatter(b2)
                            issue_gather(pbuf, ic + 2, b2)

                        cond = ((ip == _IPN - 1) if J2 == 0
                                else ((ip == _IPN - 1) & (jp < _JPN - 1)))

                        @pl.when(cond)
                        def _():
                            if b == 2:
                                pltpu.make_async_copy(
                                    packed_hbm.at[pl.ds(cstart, _SUP)],
                                    npbuf, ps[nb]).wait()
                                pltpu.make_async_copy(
                                    vals_hbm.at[pl.ds(cstart, _SUP)],
                                    nvbuf, ps[nb]).wait()
                            wait_scatter(b2)
                            issue_gather(npbuf, b - 2, b2)
                return carry

            lax.fori_loop(0, _IPN, pair_body, 0)

        return super_body

    sup0 = make_super(0)
    sup1 = make_super(1)

    def outer(jp, carry):
        sup0(jp)
        sup1(jp)
        return carry

    lax.fori_loop(0, _JPN, outer, 0)

    for b in range(4):
        wait_scatter(b)
    plsc.subcore_barrier()

    dstart = s * _ZQ

    @pl.when(s < 15)
    def _():
        pltpu.sync_copy(acc.at[pl.ds(dstart, _ZQ)],
                        out_hbm.at[c, pl.ds(dstart, _ZQ)])

    @pl.when(s == 15)
    def _():
        pltpu.sync_copy(acc.at[pl.ds(dstart, 2960)],
                        out_hbm.at[c, pl.ds(dstart, 2960)])


@functools.cache
def _make_segsum_call():
  return pl.kernel(
    _segsum_body,
    out_type=jax.ShapeDtypeStruct((2, N_NODE, _D), jnp.float32),
    mesh=plsc.VectorSubcoreMesh(core_axis_name="c", subcore_axis_name="s",
                                num_cores=2, num_subcores=16),
    compiler_params=pltpu.CompilerParams(use_tc_tiling_on_sc=False),
    scratch_types=[
        pltpu.VMEM_SHARED((_ACC_ROWS, _D), jnp.float32),
        pltpu.VMEM((_SUP, 2, _CH), jnp.int32),
        pltpu.VMEM((_SUP, 2, _CH), jnp.int32),
        pltpu.VMEM((_SUP, 1, _CH), jnp.float32),
        pltpu.VMEM((_SUP, 1, _CH), jnp.float32),
        pltpu.VMEM((_CH, _D), jnp.float32),
        pltpu.VMEM((_CH, _D), jnp.float32),
        pltpu.VMEM((_CH, _D), jnp.float32),
        pltpu.VMEM((_CH, _D), jnp.float32),
        pltpu.VMEM((_CH,), jnp.int32),
        pltpu.VMEM((_CH,), jnp.int32),
        pltpu.VMEM((_CH,), jnp.int32),
        pltpu.VMEM((_CH,), jnp.int32),
        pltpu.SemaphoreType.DMA,
        pltpu.SemaphoreType.DMA,
        pltpu.SemaphoreType.DMA,
        pltpu.SemaphoreType.DMA,
        pltpu.SemaphoreType.DMA,
        pltpu.SemaphoreType.DMA,
        pltpu.SemaphoreType.DMA,
        pltpu.SemaphoreType.DMA,
        pltpu.SemaphoreType.DMA,
        pltpu.SemaphoreType.DMA,
    ],
  )


def _pack_edges(adj_row, adj_col, adj_val):
    pad = _E_PAD - N_EDGES
    r = jnp.concatenate([adj_row, jnp.zeros((pad,), adj_row.dtype)])
    col = jnp.concatenate([adj_col, jnp.zeros((pad,), adj_col.dtype)])
    v = jnp.concatenate([adj_val, jnp.zeros((pad,), adj_val.dtype)])
    packed = jnp.stack([col.reshape(_NCHUNK, _CH).astype(jnp.int32),
                        r.reshape(_NCHUNK, _CH).astype(jnp.int32)], axis=1)
    return packed, v.reshape(_NCHUNK, 1, _CH)


def _segsum(h0, h1, packed, vals):
    return _make_segsum_call()(h0, h1, packed, vals)


def _l2norm(x, axis):
    n = jnp.sqrt(jnp.sum(x * x, axis=axis, keepdims=True))
    return x / jnp.maximum(n, 1e-12)


_BLK = 2000


def _mm(a, b):
    return lax.dot_general(a, b, (((1,), (0,)), ((), ())),
                           preferred_element_type=jnp.float32)


def _pre_body(e_ref, img_ref, txt_ref, wpi_ref, bpi_ref, wpt_ref, bpt_ref,
              w0_ref, pi_ref, pt_ref, h1a_ref, h1b_ref, gi_ref, gt_ref):
    i = pl.program_id(0)
    e = e_ref[...]
    pi = _mm(e, wpi_ref[...]) + bpi_ref[...][None, :]
    pt = _mm(e, wpt_ref[...]) + bpt_ref[...][None, :]
    pi_ref[...] = pi
    pt_ref[...] = pt
    h1 = _mm(e, w0_ref[...])
    h1a_ref[...] = h1[:, :_D]
    h1b_ref[...] = h1[:, _D:]
    gi_p = lax.dot_general(pi, img_ref[...], (((0,), (0,)), ((), ())),
                           preferred_element_type=jnp.float32)
    gt_p = lax.dot_general(pt, txt_ref[...], (((0,), (0,)), ((), ())),
                           preferred_element_type=jnp.float32)

    @pl.when(i == 0)
    def _():
        gi_ref[...] = jnp.zeros_like(gi_ref)
        gt_ref[...] = jnp.zeros_like(gt_ref)

    gi_ref[...] += gi_p
    gt_ref[...] += gt_p


def _pre_stage(embedding, image_table, text_table, prompt_img_W, prompt_img_b,
               prompt_txt_W, prompt_txt_b, w_item0):
    nblk = pl.BlockSpec((_BLK, EMB), lambda i: (i, 0))
    hblk = pl.BlockSpec((_BLK, _D), lambda i: (i, 0))
    wspec = pl.BlockSpec((EMB, EMB), lambda i: (0, 0))
    bspec = pl.BlockSpec((EMB,), lambda i: (0,))
    gspec = pl.BlockSpec((EMB, EMB), lambda i: (0, 0))
    return pl.pallas_call(
        _pre_body,
        grid=(N_NODE // _BLK,),
        in_specs=[nblk, nblk, nblk, wspec, bspec, wspec, bspec, wspec],
        out_specs=[nblk, nblk, hblk, hblk, gspec, gspec],
        out_shape=[
            jax.ShapeDtypeStruct((N_NODE, EMB), jnp.float32),
            jax.ShapeDtypeStruct((N_NODE, EMB), jnp.float32),
            jax.ShapeDtypeStruct((N_NODE, _D), jnp.float32),
            jax.ShapeDtypeStruct((N_NODE, _D), jnp.float32),
            jax.ShapeDtypeStruct((EMB, EMB), jnp.float32),
            jax.ShapeDtypeStruct((EMB, EMB), jnp.float32),
        ],
    )(embedding, image_table, text_table, prompt_img_W, prompt_img_b,
      prompt_txt_W, prompt_txt_b, w_item0)


def _mid_body(seg1_ref, w1_ref, h2a_ref, h2b_ref, n1_ref):
    segb = seg1_ref[...]
    seg1 = jnp.concatenate([segb[0], segb[1]], axis=-1)
    h2 = _mm(seg1, w1_ref[...])
    h2a_ref[...] = h2[:, :_D]
    h2b_ref[...] = h2[:, _D:]
    n1_ref[...] = _l2norm(seg1, -1)


def _mid_stage(seg1, w_item1):
    nblk = pl.BlockSpec((_BLK, EMB), lambda i: (i, 0))
    hblk = pl.BlockSpec((_BLK, _D), lambda i: (i, 0))
    sblk = pl.BlockSpec((2, _BLK, _D), lambda i: (0, i, 0))
    wspec = pl.BlockSpec((EMB, EMB), lambda i: (0, 0))
    return pl.pallas_call(
        _mid_body,
        grid=(N_NODE // _BLK,),
        in_specs=[sblk, wspec],
        out_specs=[hblk, hblk, nblk],
        out_shape=[
            jax.ShapeDtypeStruct((N_NODE, _D), jnp.float32),
            jax.ShapeDtypeStruct((N_NODE, _D), jnp.float32),
            jax.ShapeDtypeStruct((N_NODE, EMB), jnp.float32),
        ],
    )(seg1, w_item1)


def _mix_body(e_ref, n1_ref, seg2_ref, pi_ref, pt_ref, img_ref, txt_ref,
              gi_ref, gt_ref, w1_ref, b1_ref, w2_ref, b2_ref, mixed_ref):
    segb = seg2_ref[...]
    seg2 = jnp.concatenate([segb[0], segb[1]], axis=-1)
    item = (e_ref[...] + n1_ref[...] + _l2norm(seg2, -1)) * (1.0 / 3.0)
    re_img = img_ref[...] + SOFT_LAMBDA * _l2norm(_mm(pi_ref[...], gi_ref[...]), -1)
    re_txt = txt_ref[...] + SOFT_LAMBDA * _l2norm(_mm(pt_ref[...], gt_ref[...]), -1)
    w1 = w1_ref[...]
    z = (_mm(item, w1[0:EMB]) + _mm(re_img, w1[EMB:2 * EMB])
         + _mm(re_txt, w1[2 * EMB:3 * EMB]) + b1_ref[...][None, :])
    z = jnp.tanh(z)
    mixed_ref[...] = jnp.tanh(_mm(z, w2_ref[...]) + b2_ref[...][None, :])


def _mix_stage(embedding, n1, seg2, pi, pt, image_table, text_table, gi, gt,
               mlp1_W, mlp1_b, mlp2_W, mlp2_b):
    nblk = pl.BlockSpec((_BLK, EMB), lambda i: (i, 0))
    sblk = pl.BlockSpec((2, _BLK, _D), lambda i: (0, i, 0))
    gspec = pl.BlockSpec((EMB, EMB), lambda i: (0, 0))
    w1spec = pl.BlockSpec((3 * EMB, EMB), lambda i: (0, 0))
    bspec = pl.BlockSpec((EMB,), lambda i: (0,))
    return pl.pallas_call(
        _mix_body,
        grid=(N_NODE // _BLK,),
        in_specs=[nblk, nblk, sblk, nblk, nblk, nblk, nblk, gspec, gspec,
                  w1spec, bspec, gspec, bspec],
        out_specs=nblk,
        out_shape=jax.ShapeDtypeStruct((N_NODE, EMB), jnp.float32),
    )(embedding, n1, seg2, pi, pt, image_table, text_table, gi, gt,
      mlp1_W, mlp1_b, mlp2_W, mlp2_b)


def _scores_body(select_ref, mixed_ref, out_ref):
    out_ref[...] = W_K * jax.lax.dot_general(
        select_ref[...], mixed_ref[...],
        (((1,), (1,)), ((), ())),
        preferred_element_type=jnp.float32)


def _scores_matmul(select, mixed):
    blk = 2048
    grid = (pl.cdiv(N_NODE, blk),)
    return pl.pallas_call(
        _scores_body,
        grid=grid,
        in_specs=[
            pl.BlockSpec((BATCH, EMB), lambda i: (0, 0)),
            pl.BlockSpec((blk, EMB), lambda i: (i, 0)),
        ],
        out_specs=pl.BlockSpec((BATCH, blk), lambda i: (0, i)),
        out_shape=jax.ShapeDtypeStruct((BATCH, N_NODE), jnp.float32),
    )(select, mixed)


def kernel(session_item, session_len, reversed_sess_item, mask, tar, adj_row, adj_col, adj_val, embedding, pos_embedding, image_table, text_table, w_item0, w_item1, w_1, w_2, glu1_W, glu1_b, glu2_W, prompt_img_W, prompt_img_b, prompt_txt_W, prompt_txt_b, mlp1_W, mlp1_b, mlp2_W, mlp2_b):
    pi, pt, h1a, h1b, gi, gt = _pre_stage(
        embedding, image_table, text_table, prompt_img_W, prompt_img_b,
        prompt_txt_W, prompt_txt_b, w_item0)
    packed, vals = _pack_edges(adj_row, adj_col, adj_val)
    seg1 = _segsum(h1a, h1b, packed, vals)
    h2a, h2b, n1 = _mid_stage(seg1, w_item1)
    seg2 = _segsum(h2a, h2b, packed, vals)
    mixed = _mix_stage(embedding, n1, seg2, pi, pt, image_table, text_table,
                       gi, gt, mlp1_W, mlp1_b, mlp2_W, mlp2_b)
    table = jnp.concatenate([jnp.zeros((1, EMB), jnp.float32), mixed], axis=0)
    seq_h = jnp.take(table, reversed_sess_item, axis=0)
    hs = jnp.sum(seq_h, axis=1) / session_len
    m = mask.astype(jnp.float32)[..., None]
    pos = jnp.broadcast_to(pos_embedding[:SEQ][None, :, :], seq_h.shape)
    nh = jnp.tanh(jnp.concatenate([pos, seq_h], axis=-1) @ w_1)
    nh = jax.nn.sigmoid(nh @ glu1_W + glu1_b + hs[:, None, :] @ glu2_W)
    beta = (nh @ w_2) * m
    select = jnp.sum(beta * seq_h, axis=1)
    return _scores_matmul(select, mixed)
